# Initial kernel scaffold; baseline (speedup 1.0000x reference)
import jax
import jax.numpy as jnp
from jax.experimental import pallas as pl


def kernel(state, W_qkv, b_qkv, W_o, b_o, ln1_g, ln1_b, W_ff1, b_ff1, W_ff2, b_ff2, ln2_g, ln2_b, W_tc, b_tc, W_oc, b_oc, W_type, b_type, W_opcnt, b_opcnt, W_skip, b_skip, W_conv_kernel_size, b_conv_kernel_size, W_conv_out_channels, b_conv_out_channels, W_conv_stride, b_conv_stride, W_linear_out_features, b_linear_out_features, W_mha_num_heads, b_mha_num_heads, W_mha_dropout, b_mha_dropout, W_activate_fn, b_activate_fn, W_dropout_p, b_dropout_p, W_bn_momentum, b_bn_momentum):
    raise NotImplementedError("write your pallas kernel here")


# speedup vs baseline: 17.5395x; 17.5395x over previous
import functools
import math

import jax
import jax.numpy as jnp
import numpy as np
from jax import lax
from jax.experimental import pallas as pl
from jax.experimental.pallas import tpu as pltpu

S, B, D, NHEAD, FF, L = 2048, 4, 1024, 2, 1024, 12
DH = D // NHEAD
_ARCH_NS = [8, 4, 4, 5, 3, 5, 6, 3, 4]
_HEADW = 64 + 32 + 8 * len(_ARCH_NS)
_HP = jax.lax.Precision.HIGHEST


def _tf2x32(k1, k2, x1, x2):
    def rotl(x, d):
        return ((x << np.uint32(d)) | (x >> np.uint32(32 - d))).astype(np.uint32)
    ks = [np.uint32(k1), np.uint32(k2),
          np.uint32(k1) ^ np.uint32(k2) ^ np.uint32(0x1BD11BDA)]
    rots = [[13, 15, 26, 6], [17, 29, 16, 24]]
    x0 = (x1 + ks[0]).astype(np.uint32)
    y = (x2 + ks[1]).astype(np.uint32)
    for i in range(5):
        for r in rots[i % 2]:
            x0 = (x0 + y).astype(np.uint32)
            y = rotl(y, r)
            y = x0 ^ y
        x0 = (x0 + ks[(i + 1) % 3]).astype(np.uint32)
        y = (y + ks[(i + 2) % 3] + np.uint32(i + 1)).astype(np.uint32)
    return x0, y


def _np_split(key):
    b1, b2 = _tf2x32(key[0], key[1], np.zeros(2, np.uint32),
                     np.arange(2, dtype=np.uint32))
    return np.stack([b1, b2], -1)


def _np_gumbel(key, shape):
    n = int(np.prod(shape))
    idx = np.arange(n, dtype=np.uint64)
    c1 = (idx >> np.uint64(32)).astype(np.uint32)
    c2 = (idx & np.uint64(0xFFFFFFFF)).astype(np.uint32)
    b1, b2 = _tf2x32(key[0], key[1], c1, c2)
    bits = (b1 ^ b2).reshape(shape)
    fb = (bits >> np.uint32(9)) | np.uint32(0x3F800000)
    u = fb.view(np.float32) - np.float32(1.0)
    tiny = np.float32(np.finfo(np.float32).tiny)
    u = np.maximum(tiny, u * (np.float32(1.0) - tiny) + tiny)
    return -np.log(-np.log(u))


def _build_noise():
    key = np.array([0, 42], np.uint32)
    gts, gos, gss, gas = [], [], [], []

    def nxt():
        nonlocal key
        ks = _np_split(key)
        key = ks[0]
        return ks[1]

    def pad(a, w):
        return np.pad(a, ((0, 0), (0, w - a.shape[1])))

    for i in range(L):
        gts.append(_np_gumbel(nxt(), (B, 8)))
        gos.append(pad(_np_gumbel(nxt(), (B, 6)), 8))
        gss.append(pad(_np_gumbel(nxt(), (B, L - i)), L))
        for n in _ARCH_NS:
            gas.append(pad(_np_gumbel(nxt(), (B, n)), 8))
    return (np.concatenate(gts, 0), np.concatenate(gos, 0),
            np.concatenate(gss, 0), np.concatenate(gas, 0))


_NOISE = _build_noise()


def _prep_kernel(state0_ref, wqkv_ref, bqkv_ref, u_ref):
    s0 = state0_ref[...]
    q0 = jnp.dot(s0, wqkv_ref[:, :D],
                 preferred_element_type=jnp.float32, precision=_HP) + bqkv_ref[0:1, :D]
    scale = 1.0 / math.sqrt(DH)
    cols = []
    for b in range(B):
        for h in range(NHEAD):
            qh = q0[b:b + 1, DH * h:DH * (h + 1)]
            wk_h = wqkv_ref[:, D + DH * h:D + DH * (h + 1)]
            cols.append(jnp.dot(wk_h, qh.T,
                                preferred_element_type=jnp.float32, precision=_HP) * scale)
    u_ref[...] = jnp.concatenate(cols, axis=1)


_BLK = 256
_NBLK = S // _BLK


def _attn_stream_kernel(state_ref, u_ref, ws_ref, m_ref, d_ref, acc_ref):
    i = pl.program_id(0)

    @pl.when(i == 0)
    def _init():
        m_ref[...] = jnp.full((1, B * NHEAD), -1e30, jnp.float32)
        d_ref[...] = jnp.zeros((1, B * NHEAD), jnp.float32)
        acc_ref[...] = jnp.zeros((B * NHEAD, D), jnp.float32)

    uv = u_ref[...]
    sc_list = []
    st_list = []
    for b in range(B):
        st_b = state_ref[:, b, :]
        st_list.append(st_b)
        sc_list.append(jnp.dot(st_b, uv[:, 2 * b:2 * b + 2],
                               preferred_element_type=jnp.float32))
    sc = jnp.concatenate(sc_list, axis=1)
    m_old = m_ref[...]
    m_new = jnp.maximum(m_old, jnp.max(sc, axis=0, keepdims=True))
    corr = jnp.exp(m_old - m_new)
    p = jnp.exp(sc - m_new)
    m_ref[...] = m_new
    d_ref[...] = d_ref[...] * corr + jnp.sum(p, axis=0, keepdims=True)
    p_t = p.T
    upd = []
    for b in range(B):
        upd.append(jnp.dot(p_t[2 * b:2 * b + 2, :], st_list[b],
                           preferred_element_type=jnp.float32))
    acc_ref[...] = acc_ref[...] * corr.reshape(B * NHEAD, 1) + \
        jnp.concatenate(upd, axis=0)

    @pl.when(i == _NBLK - 1)
    def _fin():
        ws_ref[...] = acc_ref[...] / d_ref[...].reshape(B * NHEAD, 1)


def _post_kernel(ws_ref, s0_ref, wv_ref, bv_ref, wo_ref, bo_ref,
                 g1_ref, b1_ref, wff1_ref, bff1_ref, wff2_ref, bff2_ref,
                 g2_ref, b2_ref, wheads_ref, bheads_ref, out_ref):
    ws = ws_ref[...]
    o_parts = []
    for h in range(NHEAD):
        wsh = jnp.concatenate([ws[2 * b + h:2 * b + h + 1, :] for b in range(B)],
                              axis=0)
        o_parts.append(jnp.dot(wsh, wv_ref[:, DH * h:DH * (h + 1)],
                               preferred_element_type=jnp.float32, precision=_HP))
    o0 = jnp.concatenate(o_parts, axis=1) + bv_ref[...]

    def ln(x, g, b):
        mu = jnp.mean(x, axis=1, keepdims=True)
        v = jnp.mean((x - mu) ** 2, axis=1, keepdims=True)
        return (x - mu) / jnp.sqrt(v + 1e-5) * g + b

    y0 = s0_ref[...] + jnp.dot(o0, wo_ref[...],
                               preferred_element_type=jnp.float32, precision=_HP) + bo_ref[...]
    x1 = ln(y0, g1_ref[...], b1_ref[...])
    h1 = jnp.maximum(jnp.dot(x1, wff1_ref[...],
                             preferred_element_type=jnp.float32, precision=_HP) + bff1_ref[...], 0.0)
    ffo = jnp.dot(h1, wff2_ref[...],
                  preferred_element_type=jnp.float32, precision=_HP) + bff2_ref[...]
    hid = ln(x1 + ffo, g2_ref[...], b2_ref[...])
    out_ref[...] = jnp.dot(hid, wheads_ref[...],
                           preferred_element_type=jnp.float32, precision=_HP) + bheads_ref[...]


def _lsm(x):
    m = jnp.max(x, axis=1, keepdims=True)
    e = jnp.exp(x - m)
    return x - (m + jnp.log(jnp.sum(e, axis=1, keepdims=True)))


def _pick(lp, g):
    z = lp + g
    m = jnp.max(z, axis=1, keepdims=True)
    lane = lax.broadcasted_iota(jnp.int32, z.shape, 1)
    idx = jnp.min(jnp.where(z >= m, lane, 9999), axis=1, keepdims=True)
    return jnp.sum(jnp.where(lane == idx, lp, 0.0), axis=1, keepdims=True)


def _sample_kernel(headl_ref, wt_ref, bt_ref, wo2_ref, bo2_ref, wsk_ref, bsk_ref,
                   gt_ref, go_ref, gs_ref, ga_ref, out_ref):
    tcom = headl_ref[:, 0:64]
    ocom = headl_ref[:, 64:96]
    TL = jnp.dot(tcom, wt_ref[...],
                 preferred_element_type=jnp.float32, precision=_HP) + bt_ref[...]
    OL = jnp.dot(ocom, wo2_ref[...],
                 preferred_element_type=jnp.float32, precision=_HP) + bo2_ref[...]
    SL = jnp.dot(tcom, wsk_ref[...],
                 preferred_element_type=jnp.float32, precision=_HP) + bsk_ref[...]
    alps = []
    for k, n in enumerate(_ARCH_NS):
        off = 96 + 8 * k
        alps.append(_lsm(headl_ref[:, off:off + n]))
    total = jnp.zeros((B, 1), jnp.float32)
    for i in range(L):
        tlp = _lsm(TL[:, 8 * i:8 * i + 8])
        total += _pick(tlp, gt_ref[4 * i:4 * i + 4, :])
        olp = _lsm(OL[:, 6 * i:6 * i + 6])
        total += _pick(olp, go_ref[4 * i:4 * i + 4, :6])
        w = L - i
        slp = _lsm(SL[:, 12 * i:12 * i + w])
        total += _pick(slp, gs_ref[4 * i:4 * i + 4, :w])
        for k, n in enumerate(_ARCH_NS):
            r = (i * 9 + k) * 4
            total += _pick(alps[k], ga_ref[r:r + 4, :n])
    out_ref[...] = total


def kernel(state, W_qkv, b_qkv, W_o, b_o, ln1_g, ln1_b, W_ff1, b_ff1, W_ff2,
           b_ff2, ln2_g, ln2_b, W_tc, b_tc, W_oc, b_oc, W_type, b_type,
           W_opcnt, b_opcnt, W_skip, b_skip, W_conv_kernel_size,
           b_conv_kernel_size, W_conv_out_channels, b_conv_out_channels,
           W_conv_stride, b_conv_stride, W_linear_out_features,
           b_linear_out_features, W_mha_num_heads, b_mha_num_heads,
           W_mha_dropout, b_mha_dropout, W_activate_fn, b_activate_fn,
           W_dropout_p, b_dropout_p, W_bn_momentum, b_bn_momentum):
    f32 = jnp.float32
    gt, go, gs, ga = _NOISE

    u = pl.pallas_call(
        _prep_kernel,
        out_shape=jax.ShapeDtypeStruct((D, B * NHEAD), f32),
    )(state[0], W_qkv, b_qkv.reshape(1, -1))

    arch_ws = [W_activate_fn, W_bn_momentum, W_conv_kernel_size,
               W_conv_out_channels, W_conv_stride, W_dropout_p,
               W_linear_out_features, W_mha_dropout, W_mha_num_heads]
    arch_bs = [b_activate_fn, b_bn_momentum, b_conv_kernel_size,
               b_conv_out_channels, b_conv_stride, b_dropout_p,
               b_linear_out_features, b_mha_dropout, b_mha_num_heads]
    arch_ws = [jnp.pad(w, ((0, 0), (0, 8 - w.shape[1]))) for w in arch_ws]
    arch_bs = [jnp.pad(bb, (0, 8 - bb.shape[0])) for bb in arch_bs]
    wheads = jnp.concatenate([W_tc, W_oc] + arch_ws, axis=1)
    bheads = jnp.concatenate([b_tc, b_oc] + arch_bs).reshape(1, -1)

    ws = pl.pallas_call(
        _attn_stream_kernel,
        grid=(_NBLK,),
        in_specs=[
            pl.BlockSpec((_BLK, B, D), lambda i: (i, 0, 0)),
            pl.BlockSpec((D, B * NHEAD), lambda i: (0, 0)),
        ],
        out_specs=pl.BlockSpec((B * NHEAD, D), lambda i: (0, 0)),
        out_shape=jax.ShapeDtypeStruct((B * NHEAD, D), f32),
        scratch_shapes=[
            pltpu.VMEM((1, B * NHEAD), f32),
            pltpu.VMEM((1, B * NHEAD), f32),
            pltpu.VMEM((B * NHEAD, D), f32),
        ],
    )(state, u)

    headl = pl.pallas_call(
        _post_kernel,
        out_shape=jax.ShapeDtypeStruct((B, _HEADW), f32),
    )(ws, state[0], W_qkv[:, 2 * D:], b_qkv[2 * D:].reshape(1, -1),
      W_o, b_o.reshape(1, -1), ln1_g.reshape(1, -1), ln1_b.reshape(1, -1),
      W_ff1, b_ff1.reshape(1, -1), W_ff2, b_ff2.reshape(1, -1),
      ln2_g.reshape(1, -1), ln2_b.reshape(1, -1), wheads, bheads)

    wt = W_type.transpose(1, 0, 2).reshape(64, L * 8)
    bt = b_type.reshape(1, L * 8)
    wo2 = W_opcnt.transpose(1, 0, 2).reshape(32, L * 6)
    bo2 = b_opcnt.reshape(1, L * 6)
    wsk = W_skip.transpose(1, 0, 2).reshape(64, L * L)
    bsk = b_skip.reshape(1, L * L)

    total = pl.pallas_call(
        _sample_kernel,
        out_shape=jax.ShapeDtypeStruct((B, 1), f32),
    )(headl, wt, bt, wo2, bo2, wsk, bsk,
      jnp.asarray(gt), jnp.asarray(go), jnp.asarray(gs), jnp.asarray(ga))
    return total.reshape(B)


# speedup vs baseline: 26.9034x; 1.5339x over previous
import functools
import math

import jax
import jax.numpy as jnp
import numpy as np
from jax import lax
from jax.experimental import pallas as pl
from jax.experimental.pallas import tpu as pltpu

S, B, D, NHEAD, FF, L = 2048, 4, 1024, 2, 1024, 12
DH = D // NHEAD
_ARCH_NS = [8, 4, 4, 5, 3, 5, 6, 3, 4]
_HEADW = 64 + 32 + 8 * len(_ARCH_NS)
_HP = jax.lax.Precision.HIGHEST


def _tf2x32(k1, k2, x1, x2):
    def rotl(x, d):
        return ((x << np.uint32(d)) | (x >> np.uint32(32 - d))).astype(np.uint32)
    ks = [np.uint32(k1), np.uint32(k2),
          np.uint32(k1) ^ np.uint32(k2) ^ np.uint32(0x1BD11BDA)]
    rots = [[13, 15, 26, 6], [17, 29, 16, 24]]
    x0 = (x1 + ks[0]).astype(np.uint32)
    y = (x2 + ks[1]).astype(np.uint32)
    for i in range(5):
        for r in rots[i % 2]:
            x0 = (x0 + y).astype(np.uint32)
            y = rotl(y, r)
            y = x0 ^ y
        x0 = (x0 + ks[(i + 1) % 3]).astype(np.uint32)
        y = (y + ks[(i + 2) % 3] + np.uint32(i + 1)).astype(np.uint32)
    return x0, y


def _np_split(key):
    b1, b2 = _tf2x32(key[0], key[1], np.zeros(2, np.uint32),
                     np.arange(2, dtype=np.uint32))
    return np.stack([b1, b2], -1)


def _np_gumbel(key, shape):
    n = int(np.prod(shape))
    idx = np.arange(n, dtype=np.uint64)
    c1 = (idx >> np.uint64(32)).astype(np.uint32)
    c2 = (idx & np.uint64(0xFFFFFFFF)).astype(np.uint32)
    b1, b2 = _tf2x32(key[0], key[1], c1, c2)
    bits = (b1 ^ b2).reshape(shape)
    fb = (bits >> np.uint32(9)) | np.uint32(0x3F800000)
    u = fb.view(np.float32) - np.float32(1.0)
    tiny = np.float32(np.finfo(np.float32).tiny)
    u = np.maximum(tiny, u * (np.float32(1.0) - tiny) + tiny)
    return -np.log(-np.log(u))


def _build_noise():
    key = np.array([0, 42], np.uint32)
    gts, gos, gss, gas = [], [], [], []

    def nxt():
        nonlocal key
        ks = _np_split(key)
        key = ks[0]
        return ks[1]

    def pad(a, w):
        return np.pad(a, ((0, 0), (0, w - a.shape[1])))

    for i in range(L):
        gts.append(_np_gumbel(nxt(), (B, 8)))
        gos.append(pad(_np_gumbel(nxt(), (B, 6)), 8))
        gss.append(pad(_np_gumbel(nxt(), (B, L - i)), 16))
        for n in _ARCH_NS:
            gas.append(pad(_np_gumbel(nxt(), (B, n)), 8))
    g8 = np.concatenate(gts + gos + gas, 0)
    gs = np.concatenate(gss, 0)
    return g8, gs


_NOISE = _build_noise()
_SKIP_VALID = np.arange(16)[None, :] < (L - np.arange(L))[:, None]


def _prep_kernel(state0_ref, wqkv_ref, bqkv_ref, u_ref):
    s0 = state0_ref[...]
    q0 = jnp.dot(s0, wqkv_ref[:, :D],
                 preferred_element_type=jnp.float32, precision=_HP) + bqkv_ref[0:1, :D]
    scale = 1.0 / math.sqrt(DH)
    cols = []
    for b in range(B):
        for h in range(NHEAD):
            qh = q0[b:b + 1, DH * h:DH * (h + 1)]
            wk_h = wqkv_ref[:, D + DH * h:D + DH * (h + 1)]
            cols.append(jnp.dot(wk_h, qh.T,
                                preferred_element_type=jnp.float32, precision=_HP) * scale)
    u_ref[...] = jnp.concatenate(cols, axis=1)


_BLK = 256
_NBLK = S // _BLK


def _attn_stream_kernel(state_ref, u_ref, ws_ref, m_ref, d_ref, acc_ref):
    i = pl.program_id(0)

    @pl.when(i == 0)
    def _init():
        m_ref[...] = jnp.full((1, B * NHEAD), -1e30, jnp.float32)
        d_ref[...] = jnp.zeros((1, B * NHEAD), jnp.float32)
        acc_ref[...] = jnp.zeros((B * NHEAD, D), jnp.float32)

    uv = u_ref[...]
    sc_list = []
    st_list = []
    for b in range(B):
        st_b = state_ref[:, b, :]
        st_list.append(st_b)
        sc_list.append(jnp.dot(st_b, uv[:, 2 * b:2 * b + 2],
                               preferred_element_type=jnp.float32))
    sc = jnp.concatenate(sc_list, axis=1)
    m_old = m_ref[...]
    m_new = jnp.maximum(m_old, jnp.max(sc, axis=0, keepdims=True))
    corr = jnp.exp(m_old - m_new)
    p = jnp.exp(sc - m_new)
    m_ref[...] = m_new
    d_ref[...] = d_ref[...] * corr + jnp.sum(p, axis=0, keepdims=True)
    p_t = p.T
    upd = []
    for b in range(B):
        upd.append(jnp.dot(p_t[2 * b:2 * b + 2, :], st_list[b],
                           preferred_element_type=jnp.float32))
    acc_ref[...] = acc_ref[...] * corr.reshape(B * NHEAD, 1) + \
        jnp.concatenate(upd, axis=0)

    @pl.when(i == _NBLK - 1)
    def _fin():
        ws_ref[...] = acc_ref[...] / d_ref[...].reshape(B * NHEAD, 1)


def _post_kernel(ws_ref, s0_ref, wv_ref, bv_ref, wo_ref, bo_ref,
                 g1_ref, b1_ref, wff1_ref, bff1_ref, wff2_ref, bff2_ref,
                 g2_ref, b2_ref, wheads_ref, bheads_ref, out_ref):
    ws = ws_ref[...]
    o_parts = []
    for h in range(NHEAD):
        wsh = jnp.concatenate([ws[2 * b + h:2 * b + h + 1, :] for b in range(B)],
                              axis=0)
        o_parts.append(jnp.dot(wsh, wv_ref[:, DH * h:DH * (h + 1)],
                               preferred_element_type=jnp.float32, precision=_HP))
    o0 = jnp.concatenate(o_parts, axis=1) + bv_ref[...]

    def ln(x, g, b):
        mu = jnp.mean(x, axis=1, keepdims=True)
        v = jnp.mean((x - mu) ** 2, axis=1, keepdims=True)
        return (x - mu) / jnp.sqrt(v + 1e-5) * g + b

    y0 = s0_ref[...] + jnp.dot(o0, wo_ref[...],
                               preferred_element_type=jnp.float32, precision=_HP) + bo_ref[...]
    x1 = ln(y0, g1_ref[...], b1_ref[...])
    h1 = jnp.maximum(jnp.dot(x1, wff1_ref[...],
                             preferred_element_type=jnp.float32, precision=_HP) + bff1_ref[...], 0.0)
    ffo = jnp.dot(h1, wff2_ref[...],
                  preferred_element_type=jnp.float32, precision=_HP) + bff2_ref[...]
    hid = ln(x1 + ffo, g2_ref[...], b2_ref[...])
    out_ref[...] = jnp.dot(hid, wheads_ref[...],
                           preferred_element_type=jnp.float32, precision=_HP) + bheads_ref[...]


def _lsm(x):
    m = jnp.max(x, axis=1, keepdims=True)
    e = jnp.exp(x - m)
    return x - (m + jnp.log(jnp.sum(e, axis=1, keepdims=True)))


def _pick(lp, g):
    z = lp + g
    m = jnp.max(z, axis=1, keepdims=True)
    lane = lax.broadcasted_iota(jnp.int32, z.shape, 1)
    idx = jnp.min(jnp.where(z >= m, lane, 9999), axis=1, keepdims=True)
    return jnp.sum(jnp.where(lane == idx, lp, 0.0), axis=1, keepdims=True)


def _pickv(lp, g):
    z = lp + g
    m = jnp.max(z, axis=1, keepdims=True)
    lane = lax.broadcasted_iota(jnp.int32, z.shape, 1)
    idx = jnp.min(jnp.where(z >= m, lane, 9999), axis=1, keepdims=True)
    return jnp.sum(jnp.where(lane == idx, lp, 0.0), axis=1, keepdims=True)


def _sample_kernel(headl_ref, wt_ref, bt_ref, wo2_ref, bo2_ref, wsk_ref, bsk_ref,
                   g8_ref, gs_ref, out_ref):
    tcom = headl_ref[:, 0:64]
    ocom = headl_ref[:, 64:96]
    TL = jnp.dot(tcom, wt_ref[...],
                 preferred_element_type=jnp.float32, precision=_HP) + bt_ref[...]
    OL = jnp.dot(ocom, wo2_ref[...],
                 preferred_element_type=jnp.float32, precision=_HP) + bo2_ref[...]
    SLP = jnp.dot(tcom, wsk_ref[...],
                  preferred_element_type=jnp.float32, precision=_HP) + bsk_ref[...]
    tlb = jnp.concatenate([TL[:, 8 * i:8 * i + 8] for i in range(L)], axis=0)
    olb = jnp.concatenate([OL[:, 8 * i:8 * i + 8] for i in range(L)], axis=0)
    slb = jnp.concatenate([SLP[:, 16 * i:16 * i + 16] for i in range(L)], axis=0)
    arch_b = jnp.concatenate(
        [headl_ref[:, 96 + 8 * k:96 + 8 * k + 8] for k in range(len(_ARCH_NS))],
        axis=0)
    arch_lp = _lsm(arch_b)
    lp8 = jnp.concatenate([_lsm(tlb), _lsm(olb)] + [arch_lp] * L, axis=0)
    c8 = _pickv(lp8, g8_ref[...])
    c12 = _pickv(_lsm(slb), gs_ref[...])
    c = jnp.concatenate([c8, c12], axis=0)
    row = lax.broadcasted_iota(jnp.int32, (B, 576), 1)
    bsel = lax.broadcasted_iota(jnp.int32, (B, 576), 0)
    mask = jnp.where(row % B == bsel, 1.0, 0.0)
    out_ref[...] = jnp.dot(mask, c,
                           preferred_element_type=jnp.float32, precision=_HP)


def kernel(state, W_qkv, b_qkv, W_o, b_o, ln1_g, ln1_b, W_ff1, b_ff1, W_ff2,
           b_ff2, ln2_g, ln2_b, W_tc, b_tc, W_oc, b_oc, W_type, b_type,
           W_opcnt, b_opcnt, W_skip, b_skip, W_conv_kernel_size,
           b_conv_kernel_size, W_conv_out_channels, b_conv_out_channels,
           W_conv_stride, b_conv_stride, W_linear_out_features,
           b_linear_out_features, W_mha_num_heads, b_mha_num_heads,
           W_mha_dropout, b_mha_dropout, W_activate_fn, b_activate_fn,
           W_dropout_p, b_dropout_p, W_bn_momentum, b_bn_momentum):
    f32 = jnp.float32
    g8c, gsc = _NOISE

    u = pl.pallas_call(
        _prep_kernel,
        out_shape=jax.ShapeDtypeStruct((D, B * NHEAD), f32),
    )(state[0], W_qkv, b_qkv.reshape(1, -1))

    arch_ws = [W_activate_fn, W_bn_momentum, W_conv_kernel_size,
               W_conv_out_channels, W_conv_stride, W_dropout_p,
               W_linear_out_features, W_mha_dropout, W_mha_num_heads]
    arch_bs = [b_activate_fn, b_bn_momentum, b_conv_kernel_size,
               b_conv_out_channels, b_conv_stride, b_dropout_p,
               b_linear_out_features, b_mha_dropout, b_mha_num_heads]
    arch_ws = [jnp.pad(w, ((0, 0), (0, 8 - w.shape[1]))) for w in arch_ws]
    arch_bs = [jnp.pad(bb, (0, 8 - bb.shape[0]), constant_values=-1e30)
               for bb in arch_bs]
    wheads = jnp.concatenate([W_tc, W_oc] + arch_ws, axis=1)
    bheads = jnp.concatenate([b_tc, b_oc] + arch_bs).reshape(1, -1)

    ws = pl.pallas_call(
        _attn_stream_kernel,
        grid=(_NBLK,),
        in_specs=[
            pl.BlockSpec((_BLK, B, D), lambda i: (i, 0, 0)),
            pl.BlockSpec((D, B * NHEAD), lambda i: (0, 0)),
        ],
        out_specs=pl.BlockSpec((B * NHEAD, D), lambda i: (0, 0)),
        out_shape=jax.ShapeDtypeStruct((B * NHEAD, D), f32),
        scratch_shapes=[
            pltpu.VMEM((1, B * NHEAD), f32),
            pltpu.VMEM((1, B * NHEAD), f32),
            pltpu.VMEM((B * NHEAD, D), f32),
        ],
    )(state, u)

    headl = pl.pallas_call(
        _post_kernel,
        out_shape=jax.ShapeDtypeStruct((B, _HEADW), f32),
    )(ws, state[0], W_qkv[:, 2 * D:], b_qkv[2 * D:].reshape(1, -1),
      W_o, b_o.reshape(1, -1), ln1_g.reshape(1, -1), ln1_b.reshape(1, -1),
      W_ff1, b_ff1.reshape(1, -1), W_ff2, b_ff2.reshape(1, -1),
      ln2_g.reshape(1, -1), ln2_b.reshape(1, -1), wheads, bheads)

    wt = W_type.transpose(1, 0, 2).reshape(64, L * 8)
    bt = b_type.reshape(1, L * 8)
    wo2 = jnp.pad(W_opcnt.transpose(1, 0, 2),
                  ((0, 0), (0, 0), (0, 2))).reshape(32, L * 8)
    bo2 = jnp.pad(b_opcnt, ((0, 0), (0, 2)),
                  constant_values=-1e30).reshape(1, L * 8)
    wsk = jnp.pad(W_skip.transpose(1, 0, 2),
                  ((0, 0), (0, 0), (0, 4))).reshape(64, L * 16)
    bskp = jnp.pad(b_skip, ((0, 0), (0, 4)))
    bsk = jnp.where(_SKIP_VALID, bskp, -1e30).reshape(1, L * 16)

    total = pl.pallas_call(
        _sample_kernel,
        out_shape=jax.ShapeDtypeStruct((B, 1), f32),
    )(headl, wt, bt, wo2, bo2, wsk, bsk,
      jnp.asarray(g8c), jnp.asarray(gsc))
    return total.reshape(B)
